# baseline (device time: 63009 ns/iter reference)
import functools

import jax
import jax.numpy as jnp
import numpy as np
from jax import lax
from jax.experimental import pallas as pl
from jax.experimental.pallas import tpu as pltpu

N_DEV = 16

_RING = np.array([0, 1, 5, 9, 13, 14, 10, 6, 2, 3, 7, 11, 15, 12, 8, 4])
_INV = np.zeros(N_DEV, dtype=np.int32)
_INV[_RING] = np.arange(N_DEV, dtype=np.int32)


def _gelu(y):
    c = 0.7978845608028654
    return 0.5 * y * (1.0 + jnp.tanh(c * (y + 0.044715 * y * y * y)))


def kernel(x, w_mat):
    m, k = x.shape
    _, n = w_mat.shape
    m_out = m // N_DEV
    nh = n // 2

    ring = jnp.asarray(_RING, dtype=jnp.int32)
    my = lax.axis_index("i")
    r = jnp.asarray(_INV, dtype=jnp.int32)[my]

    meta = jnp.stack(
        [ring[jnp.mod(r + 1, N_DEV)], ring[jnp.mod(r - 1, N_DEV)]]
    ).astype(jnp.int32)

    blocks = ring[jnp.mod(r - jnp.arange(N_DEV, dtype=jnp.int32), N_DEV)]
    perm_rows = (
        blocks[:, None] * m_out + jnp.arange(m_out, dtype=jnp.int32)[None, :]
    ).reshape(-1)
    x_perm = jnp.take(x, perm_rows, axis=0)

    def body(meta_ref, x_ref, w_ref, out_ref, acc_ref,
             fcomm_ref, bcomm_ref, fsend_sems, frecv_sems, bsend_sems, brecv_sems):
        right = meta_ref[0]
        left = meta_ref[1]

        barrier = pltpu.get_barrier_semaphore()
        for nbr in (left, right):
            pl.semaphore_signal(
                barrier, inc=1, device_id=(nbr,),
                device_id_type=pl.DeviceIdType.MESH,
            )
        pl.semaphore_wait(barrier, 2)

        acc_ref[:, :] = jnp.dot(
            x_ref[:, :], w_ref[:, :], preferred_element_type=jnp.float32
        )

        prev = None
        for s in range(N_DEV - 1):
            fs_row = (s + 1) % N_DEV * m_out
            bs_row = (N_DEV - 1 - s) * m_out
            fr = pltpu.make_async_remote_copy(
                src_ref=acc_ref.at[pl.ds(fs_row, m_out), pl.ds(0, nh)],
                dst_ref=fcomm_ref.at[s],
                send_sem=fsend_sems.at[s],
                recv_sem=frecv_sems.at[s],
                device_id=(right,),
                device_id_type=pl.DeviceIdType.MESH,
            )
            br = pltpu.make_async_remote_copy(
                src_ref=acc_ref.at[pl.ds(bs_row, m_out), pl.ds(nh, nh)],
                dst_ref=bcomm_ref.at[s],
                send_sem=bsend_sems.at[s],
                recv_sem=brecv_sems.at[s],
                device_id=(left,),
                device_id_type=pl.DeviceIdType.MESH,
            )
            fr.start()
            br.start()
            if prev is not None:
                prev[0].wait_send()
                prev[1].wait_send()
            prev = (fr, br)

            fr.wait_recv()
            frow = (s + 2) % N_DEV * m_out
            acc_ref[pl.ds(frow, m_out), pl.ds(0, nh)] = (
                acc_ref[pl.ds(frow, m_out), pl.ds(0, nh)] + fcomm_ref[s, :, :]
            )
            br.wait_recv()
            brow = (N_DEV - 2 - s) * m_out
            acc_ref[pl.ds(brow, m_out), pl.ds(nh, nh)] = (
                acc_ref[pl.ds(brow, m_out), pl.ds(nh, nh)] + bcomm_ref[s, :, :]
            )
        prev[0].wait_send()
        prev[1].wait_send()

        out_ref[:, :] = _gelu(acc_ref[pl.ds(0, m_out), :])

        @functools.partial(
            pl.run_scoped, second_barrier=pltpu.SemaphoreType.REGULAR
        )
        def _(second_barrier):
            for nbr in (left, right):
                pl.semaphore_signal(
                    second_barrier, inc=1, device_id=(nbr,),
                    device_id_type=pl.DeviceIdType.MESH,
                )
            pl.semaphore_wait(second_barrier, 2)

    return pl.pallas_call(
        body,
        out_shape=jax.ShapeDtypeStruct((m_out, n), jnp.float32),
        in_specs=[
            pl.BlockSpec(memory_space=pltpu.SMEM),
            pl.BlockSpec(memory_space=pltpu.VMEM),
            pl.BlockSpec(memory_space=pltpu.VMEM),
        ],
        out_specs=pl.BlockSpec(memory_space=pltpu.VMEM),
        scratch_shapes=[
            pltpu.VMEM((m, n), jnp.float32),
            pltpu.VMEM((N_DEV - 1, m_out, nh), jnp.float32),
            pltpu.VMEM((N_DEV - 1, m_out, nh), jnp.float32),
            pltpu.SemaphoreType.DMA((N_DEV - 1,)),
            pltpu.SemaphoreType.DMA((N_DEV - 1,)),
            pltpu.SemaphoreType.DMA((N_DEV - 1,)),
            pltpu.SemaphoreType.DMA((N_DEV - 1,)),
        ],
        compiler_params=pltpu.CompilerParams(collective_id=0),
    )(meta, x_perm, w_mat)


# device time: 41928 ns/iter; 1.5028x vs baseline; 1.5028x over previous
import functools

import jax
import jax.numpy as jnp
from jax import lax
from jax.experimental import pallas as pl
from jax.experimental.pallas import tpu as pltpu

N_DEV = 16


def _gelu(y):
    c = 0.7978845608028654
    return 0.5 * y * (1.0 + jnp.tanh(c * (y + 0.044715 * y * y * y)))


def kernel(x, w_mat):
    m, k = x.shape
    _, n = w_mat.shape
    m_out = m // N_DEV

    my = lax.axis_index("i")
    d_arange = jnp.arange(N_DEV, dtype=jnp.int32)
    dests = jnp.mod(my + d_arange, N_DEV).astype(jnp.int32)

    perm_rows = (
        dests[:, None] * m_out + jnp.arange(m_out, dtype=jnp.int32)[None, :]
    ).reshape(-1)
    x_perm = jnp.take(x, perm_rows, axis=0)

    def body(meta_ref, x_ref, w_ref, out_ref, accbf_ref, comm_ref,
             send_sems, recv_sems):
        barrier = pltpu.get_barrier_semaphore()
        for d in range(1, N_DEV):
            pl.semaphore_signal(
                barrier, inc=1, device_id=(meta_ref[d],),
                device_id_type=pl.DeviceIdType.MESH,
            )
        pl.semaphore_wait(barrier, N_DEV - 1)

        accbf_ref[:, :] = jnp.dot(
            x_ref[:, :], w_ref[:, :], preferred_element_type=jnp.float32
        ).astype(jnp.bfloat16)

        rdmas = []
        for d in range(1, N_DEV):
            rdma = pltpu.make_async_remote_copy(
                src_ref=accbf_ref.at[pl.ds(d * m_out, m_out), :],
                dst_ref=comm_ref.at[d - 1],
                send_sem=send_sems.at[d - 1],
                recv_sem=recv_sems.at[d - 1],
                device_id=(meta_ref[d],),
                device_id_type=pl.DeviceIdType.MESH,
            )
            rdma.start()
            rdmas.append(rdma)

        total = accbf_ref[pl.ds(0, m_out), :].astype(jnp.float32)
        for s in range(N_DEV - 1):
            rdmas[s].wait_recv()
            total = total + comm_ref[s, :, :].astype(jnp.float32)
        out_ref[:, :] = _gelu(total)

        for rdma in rdmas:
            rdma.wait_send()

        @functools.partial(
            pl.run_scoped, second_barrier=pltpu.SemaphoreType.REGULAR
        )
        def _(second_barrier):
            for d in range(1, N_DEV):
                pl.semaphore_signal(
                    second_barrier, inc=1, device_id=(meta_ref[d],),
                    device_id_type=pl.DeviceIdType.MESH,
                )
            pl.semaphore_wait(second_barrier, N_DEV - 1)

    return pl.pallas_call(
        body,
        out_shape=jax.ShapeDtypeStruct((m_out, n), jnp.float32),
        in_specs=[
            pl.BlockSpec(memory_space=pltpu.SMEM),
            pl.BlockSpec(memory_space=pltpu.VMEM),
            pl.BlockSpec(memory_space=pltpu.VMEM),
        ],
        out_specs=pl.BlockSpec(memory_space=pltpu.VMEM),
        scratch_shapes=[
            pltpu.VMEM((m, n), jnp.bfloat16),
            pltpu.VMEM((N_DEV - 1, m_out, n), jnp.bfloat16),
            pltpu.SemaphoreType.DMA((N_DEV - 1,)),
            pltpu.SemaphoreType.DMA((N_DEV - 1,)),
        ],
        compiler_params=pltpu.CompilerParams(collective_id=0),
    )(dests, x_perm, w_mat)


# device time: 31341 ns/iter; 2.0104x vs baseline; 1.3378x over previous
import jax
import jax.numpy as jnp
from jax import lax
from jax.experimental import pallas as pl
from jax.experimental.pallas import tpu as pltpu

N_DEV = 16


def _gelu(y):
    c = 0.7978845608028654
    return 0.5 * y * (1.0 + jnp.tanh(c * (y + 0.044715 * y * y * y)))


def kernel(x, w_mat):
    m, k = x.shape
    _, n = w_mat.shape
    m_out = m // N_DEV
    mh = m // 2

    my = lax.axis_index("i")
    d_arange = jnp.arange(N_DEV, dtype=jnp.int32)
    dests = jnp.mod(my + d_arange, N_DEV).astype(jnp.int32)

    perm_rows = (
        dests[:, None] * m_out + jnp.arange(m_out, dtype=jnp.int32)[None, :]
    ).reshape(-1)
    x_perm = jnp.take(x, perm_rows, axis=0)

    def body(meta_ref, x_ref, w_ref, out_ref, accbf_ref, comm_ref,
             send_sems, recv_sems):
        barrier = pltpu.get_barrier_semaphore()
        for d in range(1, N_DEV):
            pl.semaphore_signal(
                barrier, inc=1, device_id=(meta_ref[d],),
                device_id_type=pl.DeviceIdType.MESH,
            )

        accbf_ref[pl.ds(0, mh), :] = jnp.dot(
            x_ref[pl.ds(0, mh), :], w_ref[:, :],
            preferred_element_type=jnp.float32,
        ).astype(jnp.bfloat16)

        pl.semaphore_wait(barrier, N_DEV - 1)

        def send(d):
            rdma = pltpu.make_async_remote_copy(
                src_ref=accbf_ref.at[pl.ds(d * m_out, m_out), :],
                dst_ref=comm_ref.at[d - 1],
                send_sem=send_sems.at[d - 1],
                recv_sem=recv_sems.at[d - 1],
                device_id=(meta_ref[d],),
                device_id_type=pl.DeviceIdType.MESH,
            )
            rdma.start()
            return rdma

        rdmas = [send(d) for d in range(1, N_DEV // 2)]

        accbf_ref[pl.ds(mh, mh), :] = jnp.dot(
            x_ref[pl.ds(mh, mh), :], w_ref[:, :],
            preferred_element_type=jnp.float32,
        ).astype(jnp.bfloat16)

        rdmas += [send(d) for d in range(N_DEV // 2, N_DEV)]

        groups = [(0, 4), (4, 8), (8, 12), (12, 15)]
        total = accbf_ref[pl.ds(0, m_out), :].astype(jnp.float32)
        for lo, hi in groups:
            for s in range(lo, hi):
                rdmas[s].wait_recv()
            g = comm_ref[lo, :, :].astype(jnp.float32)
            for s in range(lo + 1, hi):
                g = g + comm_ref[s, :, :].astype(jnp.float32)
            total = total + g
        out_ref[:, :] = _gelu(total)

        for rdma in rdmas:
            rdma.wait_send()

    return pl.pallas_call(
        body,
        out_shape=jax.ShapeDtypeStruct((m_out, n), jnp.float32),
        in_specs=[
            pl.BlockSpec(memory_space=pltpu.SMEM),
            pl.BlockSpec(memory_space=pltpu.VMEM),
            pl.BlockSpec(memory_space=pltpu.VMEM),
        ],
        out_specs=pl.BlockSpec(memory_space=pltpu.VMEM),
        scratch_shapes=[
            pltpu.VMEM((m, n), jnp.bfloat16),
            pltpu.VMEM((N_DEV - 1, m_out, n), jnp.bfloat16),
            pltpu.SemaphoreType.DMA((N_DEV - 1,)),
            pltpu.SemaphoreType.DMA((N_DEV - 1,)),
        ],
        compiler_params=pltpu.CompilerParams(collective_id=0),
    )(dests, x_perm, w_mat)


# device time: 26951 ns/iter; 2.3379x vs baseline; 1.1629x over previous
import jax
import jax.numpy as jnp
from jax import lax
from jax.experimental import pallas as pl
from jax.experimental.pallas import tpu as pltpu

N_DEV = 16


def _gelu(y):
    c = 0.7978845608028654
    return 0.5 * y * (1.0 + jnp.tanh(c * (y + 0.044715 * y * y * y)))


def kernel(x, w_mat):
    m, k = x.shape
    _, n = w_mat.shape
    m_out = m // N_DEV
    mh = m // 2

    def body(x_ref, w_ref, out_ref, accbf_ref, comm_ref,
             send_sems, recv_sems):
        my = lax.axis_index("i")

        barrier = pltpu.get_barrier_semaphore()
        for b in range(N_DEV):
            @pl.when(b != my)
            def _():
                pl.semaphore_signal(
                    barrier, inc=1, device_id=(b,),
                    device_id_type=pl.DeviceIdType.MESH,
                )

        accbf_ref[pl.ds(0, mh), :] = jnp.dot(
            x_ref[pl.ds(0, mh), :], w_ref[:, :],
            preferred_element_type=jnp.float32,
        ).astype(jnp.bfloat16)

        pl.semaphore_wait(barrier, N_DEV - 1)

        def send(b):
            @pl.when(b != my)
            def _():
                q = jnp.mod(b - my - 1, N_DEV)
                rdma = pltpu.make_async_remote_copy(
                    src_ref=accbf_ref.at[pl.ds(b * m_out, m_out), :],
                    dst_ref=comm_ref.at[q],
                    send_sem=send_sems.at[b],
                    recv_sem=recv_sems.at[q],
                    device_id=(b,),
                    device_id_type=pl.DeviceIdType.MESH,
                )
                rdma.start()

        for b in range(N_DEV // 2):
            send(b)

        accbf_ref[pl.ds(mh, mh), :] = jnp.dot(
            x_ref[pl.ds(mh, mh), :], w_ref[:, :],
            preferred_element_type=jnp.float32,
        ).astype(jnp.bfloat16)

        for b in range(N_DEV // 2, N_DEV):
            send(b)

        def recv_wait(s):
            rdma = pltpu.make_async_remote_copy(
                src_ref=accbf_ref.at[pl.ds(0, m_out), :],
                dst_ref=comm_ref.at[s],
                send_sem=send_sems.at[0],
                recv_sem=recv_sems.at[s],
                device_id=(my,),
                device_id_type=pl.DeviceIdType.MESH,
            )
            rdma.wait_recv()

        myrow = pl.multiple_of(my * m_out, m_out)
        total = accbf_ref[pl.ds(myrow, m_out), :].astype(jnp.float32)
        for lo, hi in [(0, 4), (4, 8), (8, 12), (12, 15)]:
            for s in range(lo, hi):
                recv_wait(s)
            g = comm_ref[lo, :, :].astype(jnp.float32)
            for s in range(lo + 1, hi):
                g = g + comm_ref[s, :, :].astype(jnp.float32)
            total = total + g
        out_ref[:, :] = _gelu(total)

        for b in range(N_DEV):
            @pl.when(b != my)
            def _():
                rdma = pltpu.make_async_remote_copy(
                    src_ref=accbf_ref.at[pl.ds(b * m_out, m_out), :],
                    dst_ref=comm_ref.at[0],
                    send_sem=send_sems.at[b],
                    recv_sem=recv_sems.at[0],
                    device_id=(b,),
                    device_id_type=pl.DeviceIdType.MESH,
                )
                rdma.wait_send()

    return pl.pallas_call(
        body,
        out_shape=jax.ShapeDtypeStruct((m_out, n), jnp.float32),
        in_specs=[
            pl.BlockSpec(memory_space=pltpu.VMEM),
            pl.BlockSpec(memory_space=pltpu.VMEM),
        ],
        out_specs=pl.BlockSpec(memory_space=pltpu.VMEM),
        scratch_shapes=[
            pltpu.VMEM((m, n), jnp.bfloat16),
            pltpu.VMEM((N_DEV - 1, m_out, n), jnp.bfloat16),
            pltpu.SemaphoreType.DMA((N_DEV,)),
            pltpu.SemaphoreType.DMA((N_DEV - 1,)),
        ],
        compiler_params=pltpu.CompilerParams(collective_id=0),
    )(x, w_mat)
